# baseline (speedup 1.0000x reference)
import functools

import jax
import jax.numpy as jnp
from jax import lax
from jax.experimental import pallas as pl
from jax.experimental.pallas import tpu as pltpu
from jax.experimental.pallas import tpu_sc as plsc

VOCAB = 1000000
D = 64
B_TOK = 4096 * 200
NC, NS, L = 2, 16, 16
NW = NC * NS
PER_W = B_TOK // NW
CHUNK = 128
NCHUNK = PER_W // CHUNK


def _embed_kernel(table_hbm, idx_hbm, out_hbm, idx_v, rows_v, gsem, osem):
    wid = lax.axis_index("s") * NC + lax.axis_index("c")
    base = wid * PER_W

    pltpu.sync_copy(idx_hbm.at[wid], idx_v)

    def body(j, _):
        pltpu.async_copy(table_hbm.at[idx_v.at[j]], rows_v, gsem).wait()

        def scale_row(i, _):
            for k in range(D // L):
                sl = pl.ds(k * L, L)
                rows_v[i, sl] = rows_v[i, sl] * 8.0
            return 0

        lax.fori_loop(0, CHUNK, scale_row, 0)

        pltpu.async_copy(rows_v, out_hbm.at[pl.ds(base + j * CHUNK, CHUNK)],
                         osem).wait()
        return 0

    lax.fori_loop(0, NCHUNK, body, 0)


@jax.jit
def _embed(table, idx):
    mesh = plsc.VectorSubcoreMesh(core_axis_name="c", subcore_axis_name="s")
    f = functools.partial(
        pl.kernel,
        out_type=jax.ShapeDtypeStruct((B_TOK, D), jnp.float32),
        mesh=mesh,
        scratch_types=[
            pltpu.VMEM((NCHUNK, CHUNK), jnp.int32),
            pltpu.VMEM((CHUNK, D), jnp.float32),
            pltpu.SemaphoreType.DMA,
            pltpu.SemaphoreType.DMA,
        ],
        compiler_params=pltpu.CompilerParams(use_tc_tiling_on_sc=False),
    )(_embed_kernel)
    return f(table, idx)


def kernel(x, table):
    idx = x.reshape(NW, NCHUNK, CHUNK).astype(jnp.int32)
    out = _embed(table, idx)
    return out.reshape(x.shape[0], x.shape[1], D)


# speedup vs baseline: 1.2088x; 1.2088x over previous
import functools

import jax
import jax.numpy as jnp
from jax import lax
from jax.experimental import pallas as pl
from jax.experimental.pallas import tpu as pltpu
from jax.experimental.pallas import tpu_sc as plsc

VOCAB = 1000000
D = 64
B_TOK = 4096 * 200
NC, NS, L = 2, 16, 16
NW = NC * NS
PER_W = B_TOK // NW
CHUNK = 128
NCHUNK = PER_W // CHUNK
NBUF = 8
NGROUP = NCHUNK // NBUF


def _embed_kernel(table_hbm, idx_hbm, out_hbm, idx_v, *bufs):
    rows = bufs[:NBUF]
    gsem = bufs[NBUF:2 * NBUF]
    osem = bufs[2 * NBUF:3 * NBUF]

    wid = lax.axis_index("s") * NC + lax.axis_index("c")
    base = wid * PER_W

    pltpu.sync_copy(idx_hbm.at[wid], idx_v)

    for b in range(NBUF):
        pltpu.make_async_copy(table_hbm.at[idx_v.at[b]], rows[b],
                              gsem[b]).start()

    def group(g, _):
        for b in range(NBUF):
            j = g * NBUF + b
            pltpu.make_async_copy(table_hbm.at[idx_v.at[j]], rows[b],
                                  gsem[b]).wait()

            @plsc.parallel_loop(0, CHUNK, step=4)
            def scale_row(i):
                for r in range(4):
                    for k in range(D // L):
                        sl = pl.ds(k * L, L)
                        rows[b][i + r, sl] = rows[b][i + r, sl] * 8.0

            pltpu.make_async_copy(
                rows[b], out_hbm.at[pl.ds(base + j * CHUNK, CHUNK)],
                osem[b]).start()

        for b in range(NBUF):
            j = g * NBUF + b
            pltpu.make_async_copy(
                rows[b], out_hbm.at[pl.ds(base + j * CHUNK, CHUNK)],
                osem[b]).wait()

            @pl.when(g + 1 < NGROUP)
            def _():
                jn = (g + 1) * NBUF + b
                pltpu.make_async_copy(table_hbm.at[idx_v.at[jn]], rows[b],
                                      gsem[b]).start()

        return 0

    lax.fori_loop(0, NGROUP, group, 0)


@jax.jit
def _embed(table, idx):
    mesh = plsc.VectorSubcoreMesh(core_axis_name="c", subcore_axis_name="s")
    f = functools.partial(
        pl.kernel,
        out_type=jax.ShapeDtypeStruct((B_TOK, D), jnp.float32),
        mesh=mesh,
        scratch_types=(
            [pltpu.VMEM((NCHUNK, CHUNK), jnp.int32)]
            + [pltpu.VMEM((CHUNK, D), jnp.float32) for _ in range(NBUF)]
            + [pltpu.SemaphoreType.DMA for _ in range(2 * NBUF)]
        ),
        compiler_params=pltpu.CompilerParams(use_tc_tiling_on_sc=False),
    )(_embed_kernel)
    return f(table, idx)


def kernel(x, table):
    idx = x.reshape(NW, NCHUNK, CHUNK).astype(jnp.int32)
    out = _embed(table, idx)
    return out.reshape(x.shape[0], x.shape[1], D)


# speedup vs baseline: 1.2089x; 1.0000x over previous
import functools

import jax
import jax.numpy as jnp
from jax import lax
from jax.experimental import pallas as pl
from jax.experimental.pallas import tpu as pltpu
from jax.experimental.pallas import tpu_sc as plsc

VOCAB = 1000000
D = 64
BATCH = 4096
SEQ = 200
NC, NS, L = 2, 16, 16
NW = NC * NS
ROWS_W = BATCH // NW
SPLITS = (0, 120)
WIDTHS = (120, 80)
NBUF = 8
CHUNKS_W = ROWS_W * 2
NGROUP = CHUNKS_W // NBUF


def _embed_kernel(table_hbm, x_hbm, out_hbm, idx_v, *bufs):
    rows = bufs[:NBUF]
    gsem = bufs[NBUF:2 * NBUF]
    osem = bufs[2 * NBUF:3 * NBUF]

    wid = lax.axis_index("s") * NC + lax.axis_index("c")
    row0 = wid * ROWS_W

    pltpu.sync_copy(x_hbm.at[pl.ds(row0, ROWS_W)], idx_v)

    def chunk_refs(b, r):
        off, width = SPLITS[b % 2], WIDTHS[b % 2]
        src = table_hbm.at[idx_v.at[r, pl.ds(off, width)]]
        dst = out_hbm.at[row0 + r, pl.ds(off, width)]
        return src, dst

    for b in range(NBUF):
        src, _ = chunk_refs(b, b // 2)
        pltpu.make_async_copy(src, rows[b], gsem[b]).start()

    def group(g, _):
        for b in range(NBUF):
            r = g * (NBUF // 2) + b // 2
            width = WIDTHS[b % 2]
            src, dst = chunk_refs(b, r)
            pltpu.make_async_copy(src, rows[b], gsem[b]).wait()

            @plsc.parallel_loop(0, width, step=4)
            def scale_row(i):
                for rr in range(4):
                    for k in range(D // L):
                        sl = pl.ds(k * L, L)
                        rows[b][i + rr, sl] = rows[b][i + rr, sl] * 8.0

            pltpu.make_async_copy(rows[b], dst, osem[b]).start()

        for b in range(NBUF):
            r = g * (NBUF // 2) + b // 2
            _, dst = chunk_refs(b, r)
            pltpu.make_async_copy(rows[b], dst, osem[b]).wait()

            @pl.when(g + 1 < NGROUP)
            def _():
                rn = (g + 1) * (NBUF // 2) + b // 2
                srcn, _ = chunk_refs(b, rn)
                pltpu.make_async_copy(srcn, rows[b], gsem[b]).start()

        return 0

    lax.fori_loop(0, NGROUP, group, 0)


@jax.jit
def _embed(table, x):
    mesh = plsc.VectorSubcoreMesh(core_axis_name="c", subcore_axis_name="s")
    f = functools.partial(
        pl.kernel,
        out_type=jax.ShapeDtypeStruct((BATCH, SEQ, D), jnp.float32),
        mesh=mesh,
        scratch_types=(
            [pltpu.VMEM((ROWS_W, SEQ), jnp.int32)]
            + [pltpu.VMEM((WIDTHS[b % 2], D), jnp.float32)
               for b in range(NBUF)]
            + [pltpu.SemaphoreType.DMA for _ in range(2 * NBUF)]
        ),
        compiler_params=pltpu.CompilerParams(use_tc_tiling_on_sc=False),
    )(_embed_kernel)
    return f(table, x)


def kernel(x, table):
    return _embed(table, x.astype(jnp.int32))
